# baseline (device time: 228695 ns/iter reference)
import jax
import jax.numpy as jnp
from jax import lax
from jax.experimental import pallas as pl
from jax.experimental.pallas import tpu as pltpu

M = 8192
N = 1024
H = M // 2
K = 32
CHR = H // K


def kernel(x):
    x = x.reshape(M, 2 * N)

    def body(x_hbm, dummy_hbm, out_hbm, own_v, recv_v, res_v,
             y_send_sems, y_recv_sems, x_send_sems, x_recv_sems,
             own_sem, store_sem):
        my_x = lax.axis_index("x")
        my_y = lax.axis_index("y")
        peer_y = 1 - my_y
        peer_x = 1 - my_x
        row_base = my_x * H

        barrier_sem = pltpu.get_barrier_semaphore()
        for dev in [(my_x, peer_y), (peer_x, my_y)]:
            pl.semaphore_signal(
                barrier_sem, inc=1,
                device_id=dev, device_id_type=pl.DeviceIdType.MESH,
            )
        pl.semaphore_wait(barrier_sem, 2)

        own_copy = pltpu.make_async_copy(
            x_hbm.at[pl.ds(row_base, H), pl.ds(my_y * N, N)], own_v, own_sem
        )
        own_copy.start()

        y_rdmas = []
        for k in range(K):
            r = pltpu.make_async_remote_copy(
                src_ref=x_hbm.at[pl.ds(row_base + k * CHR, CHR),
                                 pl.ds(peer_y * N, N)],
                dst_ref=recv_v.at[pl.ds(k * CHR, CHR), :],
                send_sem=y_send_sems.at[k],
                recv_sem=y_recv_sems.at[k],
                device_id=(my_x, peer_y),
                device_id_type=pl.DeviceIdType.MESH,
            )
            r.start()
            y_rdmas.append(r)

        own_copy.wait()

        x_rdmas = []
        for k in range(K):
            rows = pl.ds(k * CHR, CHR)
            y_rdmas[k].wait_recv()
            res_v[rows, :] = own_v[rows, :] + recv_v[rows, :]
            r = pltpu.make_async_remote_copy(
                src_ref=res_v.at[rows, :],
                dst_ref=out_hbm.at[pl.ds(row_base + k * CHR, CHR), :],
                send_sem=x_send_sems.at[k],
                recv_sem=x_recv_sems.at[k],
                device_id=(peer_x, my_y),
                device_id_type=pl.DeviceIdType.MESH,
            )
            r.start()
            x_rdmas.append(r)

        store = pltpu.make_async_copy(
            res_v, out_hbm.at[pl.ds(row_base, H), :], store_sem
        )
        store.start()

        for k in range(K):
            y_rdmas[k].wait_send()
            x_rdmas[k].wait_send()
            x_rdmas[k].wait_recv()
        store.wait()

    dummy = jnp.zeros((M, N), jnp.float32)
    out = pl.pallas_call(
        body,
        out_shape=jax.ShapeDtypeStruct((M, N), jnp.float32),
        in_specs=[pl.BlockSpec(memory_space=pltpu.MemorySpace.HBM),
                  pl.BlockSpec(memory_space=pltpu.MemorySpace.HBM)],
        out_specs=pl.BlockSpec(memory_space=pltpu.MemorySpace.HBM),
        input_output_aliases={1: 0},
        scratch_shapes=[
            pltpu.VMEM((H, N), jnp.float32),
            pltpu.VMEM((H, N), jnp.float32),
            pltpu.VMEM((H, N), jnp.float32),
            pltpu.SemaphoreType.DMA((K,)),
            pltpu.SemaphoreType.DMA((K,)),
            pltpu.SemaphoreType.DMA((K,)),
            pltpu.SemaphoreType.DMA((K,)),
            pltpu.SemaphoreType.DMA,
            pltpu.SemaphoreType.DMA,
        ],
        compiler_params=pltpu.CompilerParams(
            collective_id=0,
            vmem_limit_bytes=60 * 1024 * 1024,
        ),
    )(x, dummy)
    return out


# device time: 216889 ns/iter; 1.0544x vs baseline; 1.0544x over previous
import jax
import jax.numpy as jnp
from jax import lax
from jax.experimental import pallas as pl
from jax.experimental.pallas import tpu as pltpu

M = 8192
N = 1024
H = M // 2

_SIZES = [32, 32, 64] + [128] * 30 + [64, 32, 32]
assert sum(_SIZES) == H, sum(_SIZES)
_OFFS = [sum(_SIZES[:i]) for i in range(len(_SIZES))]
K = len(_SIZES)


def kernel(x):
    x = x.reshape(M, 2 * N)

    def body(x_hbm, out_hbm, own_v, recv_v, res_v,
             y_send_sems, y_recv_sems, x_send_sems, x_recv_sems,
             own_sem, store_sem):
        my_x = lax.axis_index("x")
        my_y = lax.axis_index("y")
        peer_y = 1 - my_y
        peer_x = 1 - my_x
        row_base = my_x * H

        barrier_sem = pltpu.get_barrier_semaphore()
        for dev in [(my_x, peer_y), (peer_x, my_y)]:
            pl.semaphore_signal(
                barrier_sem, inc=1,
                device_id=dev, device_id_type=pl.DeviceIdType.MESH,
            )
        pl.semaphore_wait(barrier_sem, 2)

        own_copy = pltpu.make_async_copy(
            x_hbm.at[pl.ds(row_base, H), pl.ds(my_y * N, N)], own_v, own_sem
        )
        own_copy.start()

        y_rdmas = []
        for k in range(K):
            r = pltpu.make_async_remote_copy(
                src_ref=x_hbm.at[pl.ds(row_base + _OFFS[k], _SIZES[k]),
                                 pl.ds(peer_y * N, N)],
                dst_ref=recv_v.at[pl.ds(_OFFS[k], _SIZES[k]), :],
                send_sem=y_send_sems.at[k],
                recv_sem=y_recv_sems.at[k],
                device_id=(my_x, peer_y),
                device_id_type=pl.DeviceIdType.MESH,
            )
            r.start()
            y_rdmas.append(r)

        own_copy.wait()

        x_rdmas = []
        for k in range(K):
            rows = pl.ds(_OFFS[k], _SIZES[k])
            y_rdmas[k].wait_recv()
            res_v[rows, :] = own_v[rows, :] + recv_v[rows, :]
            r = pltpu.make_async_remote_copy(
                src_ref=res_v.at[rows, :],
                dst_ref=out_hbm.at[pl.ds(row_base + _OFFS[k], _SIZES[k]), :],
                send_sem=x_send_sems.at[k],
                recv_sem=x_recv_sems.at[k],
                device_id=(peer_x, my_y),
                device_id_type=pl.DeviceIdType.MESH,
            )
            r.start()
            x_rdmas.append(r)

        store = pltpu.make_async_copy(
            res_v, out_hbm.at[pl.ds(row_base, H), :], store_sem
        )
        store.start()

        for k in range(K):
            y_rdmas[k].wait_send()
            x_rdmas[k].wait_send()
            x_rdmas[k].wait_recv()
        store.wait()

    out = pl.pallas_call(
        body,
        out_shape=jax.ShapeDtypeStruct((M, N), jnp.float32),
        in_specs=[pl.BlockSpec(memory_space=pltpu.MemorySpace.HBM)],
        out_specs=pl.BlockSpec(memory_space=pltpu.MemorySpace.HBM),
        scratch_shapes=[
            pltpu.VMEM((H, N), jnp.float32),
            pltpu.VMEM((H, N), jnp.float32),
            pltpu.VMEM((H, N), jnp.float32),
            pltpu.SemaphoreType.DMA((K,)),
            pltpu.SemaphoreType.DMA((K,)),
            pltpu.SemaphoreType.DMA((K,)),
            pltpu.SemaphoreType.DMA((K,)),
            pltpu.SemaphoreType.DMA,
            pltpu.SemaphoreType.DMA,
        ],
        compiler_params=pltpu.CompilerParams(
            collective_id=0,
            vmem_limit_bytes=60 * 1024 * 1024,
        ),
    )(x)
    return out
